# baseline (speedup 1.0000x reference)
import functools

import jax
import jax.numpy as jnp
from jax import lax
from jax.experimental import pallas as pl
from jax.experimental.pallas import tpu as pltpu
from jax.experimental.pallas import tpu_sc as plsc

BATCH = 1024
SEQ = 200
DIM = 128
LANES = 16
CHUNK = 40
NCHUNK = SEQ // CHUNK


def _emb_body(x_hbm, pos_hbm, tok_hbm, out_hbm, pos_v, idx_v, rows_v, gsem):
    info = plsc.get_sparse_core_info()
    nc, ns = info.num_cores, info.num_subcores
    wid = lax.axis_index("s") * nc + lax.axis_index("c")
    per = BATCH // (nc * ns)

    pltpu.sync_copy(pos_hbm, pos_v)

    def seq_body(i, carry):
        b = wid * per + i
        pltpu.sync_copy(x_hbm.at[b], idx_v)
        cps = [
            pltpu.async_copy(
                tok_hbm.at[idx_v.at[pl.ds(c * CHUNK, CHUNK)]],
                rows_v.at[pl.ds(c * CHUNK, CHUNK)],
                gsem,
            )
            for c in range(NCHUNK)
        ]
        for c in range(NCHUNK):
            cps[c].wait()
            base = c * CHUNK

            def row_body(r, c2, base=base):
                for k in range(DIM // LANES):
                    v = pos_v[base + r, pl.ds(k * LANES, LANES)]
                    plsc.addupdate(
                        rows_v.at[base + r, pl.ds(k * LANES, LANES)], v
                    )
                return c2

            lax.fori_loop(0, CHUNK, row_body, 0)
        pltpu.sync_copy(rows_v, out_hbm.at[b])
        return carry

    lax.fori_loop(0, per, seq_body, 0)


@jax.jit
def _emb(x, pos_table, token_table):
    mesh = plsc.VectorSubcoreMesh(core_axis_name="c", subcore_axis_name="s")
    fn = functools.partial(
        pl.kernel,
        mesh=mesh,
        out_type=jax.ShapeDtypeStruct((BATCH, SEQ, DIM), jnp.float32),
        scratch_types=[
            pltpu.VMEM((SEQ, DIM), jnp.float32),
            pltpu.VMEM((SEQ,), jnp.int32),
            pltpu.VMEM((SEQ, DIM), jnp.float32),
            pltpu.SemaphoreType.DMA,
        ],
    )(_emb_body)
    return fn(x, pos_table, token_table)


def kernel(x, pos_table, token_table):
    return _emb(x.astype(jnp.int32), pos_table, token_table)


# speedup vs baseline: 1.6532x; 1.6532x over previous
import functools

import jax
import jax.numpy as jnp
from jax import lax
from jax.experimental import pallas as pl
from jax.experimental.pallas import tpu as pltpu
from jax.experimental.pallas import tpu_sc as plsc

BATCH = 1024
SEQ = 200
DIM = 128
LANES = 16
CHUNK = 40
NCHUNK = SEQ // CHUNK


def _emb_body(
    x_hbm, pos_hbm, tok_hbm, out_hbm, pos_v, idx_v, rows_v,
    g0, g1, g2, g3, g4, osem
):
    info = plsc.get_sparse_core_info()
    nc, ns = info.num_cores, info.num_subcores
    wid = lax.axis_index("s") * nc + lax.axis_index("c")
    per = BATCH // (nc * ns)
    base_b = wid * per
    gsems = (g0, g1, g2, g3, g4)

    pltpu.sync_copy(pos_hbm, pos_v)
    pltpu.sync_copy(x_hbm.at[pl.ds(base_b, per)], idx_v)

    def issue(i, slot):
        for c in range(NCHUNK):
            pltpu.async_copy(
                tok_hbm.at[idx_v.at[i].at[c]],
                rows_v.at[slot].at[pl.ds(c * CHUNK, CHUNK)],
                gsems[c],
            )

    def process(i, slot):
        for c in range(NCHUNK):
            pltpu.make_async_copy(
                tok_hbm.at[pl.ds(0, CHUNK)],
                rows_v.at[slot].at[pl.ds(c * CHUNK, CHUNK)],
                gsems[c],
            ).wait()
            base = c * CHUNK

            def row_body(r, carry, base=base, slot=slot):
                for k in range(DIM // LANES):
                    v = pos_v[base + r, pl.ds(k * LANES, LANES)]
                    plsc.addupdate(
                        rows_v.at[slot].at[base + r, pl.ds(k * LANES, LANES)], v
                    )
                return carry

            lax.fori_loop(0, CHUNK, row_body, 0)
            pltpu.async_copy(
                rows_v.at[slot].at[pl.ds(base, CHUNK)],
                out_hbm.at[base_b + i].at[pl.ds(base, CHUNK)],
                osem,
            )

    def drain_wb(slot):
        pltpu.make_async_copy(
            tok_hbm.at[pl.ds(0, SEQ)], rows_v.at[slot], osem
        ).wait()

    issue(0, 0)

    def jbody(j, carry):
        a = 2 * j
        issue(a + 1, 1)
        process(a, 0)
        drain_wb(0)

        @pl.when(j < per // 2 - 1)
        def _():
            issue(a + 2, 0)

        process(a + 1, 1)
        drain_wb(1)
        return carry

    lax.fori_loop(0, per // 2, jbody, 0)


@jax.jit
def _emb(x, pos_table, token_table):
    mesh = plsc.VectorSubcoreMesh(core_axis_name="c", subcore_axis_name="s")
    per = BATCH // 32
    fn = functools.partial(
        pl.kernel,
        mesh=mesh,
        out_type=jax.ShapeDtypeStruct((BATCH, SEQ, DIM), jnp.float32),
        scratch_types=[
            pltpu.VMEM((SEQ, DIM), jnp.float32),
            pltpu.VMEM((per, NCHUNK, CHUNK), jnp.int32),
            pltpu.VMEM((2, SEQ, DIM), jnp.float32),
            pltpu.SemaphoreType.DMA,
            pltpu.SemaphoreType.DMA,
            pltpu.SemaphoreType.DMA,
            pltpu.SemaphoreType.DMA,
            pltpu.SemaphoreType.DMA,
            pltpu.SemaphoreType.DMA,
        ],
    )(_emb_body)
    return fn(x.reshape(BATCH, NCHUNK, CHUNK), pos_table, token_table)


def kernel(x, pos_table, token_table):
    return _emb(x.astype(jnp.int32), pos_table, token_table)


# speedup vs baseline: 1.6534x; 1.0001x over previous
import functools

import jax
import jax.numpy as jnp
from jax import lax
from jax.experimental import pallas as pl
from jax.experimental.pallas import tpu as pltpu
from jax.experimental.pallas import tpu_sc as plsc

BATCH = 1024
SEQ = 200
DIM = 128
LANES = 16
CH0 = 128
CH1 = SEQ - CH0
CHUNKS = ((0, CH0), (CH0, CH1))


def _emb_body(x_hbm, pos_hbm, tok_hbm, out_hbm, pos_v, idx_v, rows_v, g0, g1, osem):
    info = plsc.get_sparse_core_info()
    nc, ns = info.num_cores, info.num_subcores
    wid = lax.axis_index("s") * nc + lax.axis_index("c")
    per = BATCH // (nc * ns)
    base_b = wid * per
    gsems = (g0, g1)

    pltpu.sync_copy(pos_hbm, pos_v)
    pltpu.sync_copy(x_hbm.at[pl.ds(base_b, per)], idx_v)

    def issue(i, slot):
        for c, (base, n) in enumerate(CHUNKS):
            pltpu.async_copy(
                tok_hbm.at[idx_v.at[i].at[pl.ds(base, n)]],
                rows_v.at[slot].at[pl.ds(base, n)],
                gsems[c],
            )

    def process(i, slot):
        for c, (base, n) in enumerate(CHUNKS):
            pltpu.make_async_copy(
                tok_hbm.at[pl.ds(0, n)],
                rows_v.at[slot].at[pl.ds(base, n)],
                gsems[c],
            ).wait()

            def row_body(r, carry, base=base, slot=slot):
                for k in range(DIM // LANES):
                    v = pos_v[base + r, pl.ds(k * LANES, LANES)]
                    plsc.addupdate(
                        rows_v.at[slot].at[base + r, pl.ds(k * LANES, LANES)], v
                    )
                return carry

            lax.fori_loop(0, n, row_body, 0)
            pltpu.async_copy(
                rows_v.at[slot].at[pl.ds(base, n)],
                out_hbm.at[base_b + i].at[pl.ds(base, n)],
                osem,
            )

    def drain_wb(slot):
        pltpu.make_async_copy(
            tok_hbm.at[pl.ds(0, SEQ)], rows_v.at[slot], osem
        ).wait()

    issue(0, 0)

    def jbody(j, carry):
        a = 2 * j
        issue(a + 1, 1)
        process(a, 0)
        drain_wb(0)

        @pl.when(j < per // 2 - 1)
        def _():
            issue(a + 2, 0)

        process(a + 1, 1)
        drain_wb(1)
        return carry

    lax.fori_loop(0, per // 2, jbody, 0)


@jax.jit
def _emb(x, pos_table, token_table):
    mesh = plsc.VectorSubcoreMesh(core_axis_name="c", subcore_axis_name="s")
    per = BATCH // 32
    fn = functools.partial(
        pl.kernel,
        mesh=mesh,
        out_type=jax.ShapeDtypeStruct((BATCH, SEQ, DIM), jnp.float32),
        scratch_types=[
            pltpu.VMEM((SEQ, DIM), jnp.float32),
            pltpu.VMEM((per, SEQ), jnp.int32),
            pltpu.VMEM((2, SEQ, DIM), jnp.float32),
            pltpu.SemaphoreType.DMA,
            pltpu.SemaphoreType.DMA,
            pltpu.SemaphoreType.DMA,
        ],
    )(_emb_body)
    return fn(x, pos_table, token_table)


def kernel(x, pos_table, token_table):
    return _emb(x.astype(jnp.int32), pos_table, token_table)
